# baseline (speedup 1.0000x reference)
import functools

import jax
import jax.numpy as jnp
from jax import lax
from jax.experimental import pallas as pl
from jax.experimental.pallas import tpu as pltpu
from jax.experimental.pallas import tpu_sc as plsc

NC = 2
NS = 16
LANES = 16
CHUNK = 128
CROWS = 8192
NBUF = 2


def _sc_segment_sum(N, EP, C, NB):
    HPAD = -(-N // (NC * NS * 8)) * (NS * 8)
    RPT = HPAD // NS

    @functools.partial(
        pl.kernel,
        out_type=(
            jax.ShapeDtypeStruct((NC, NS, RPT, C), jnp.float32),
            jax.ShapeDtypeStruct((NC, NS, CROWS // NS), jnp.float32),
        ),
        mesh=plsc.VectorSubcoreMesh(core_axis_name="c", subcore_axis_name="s"),
        scratch_types=[
            pltpu.VMEM((NB, CHUNK), jnp.int32),
            pltpu.VMEM((NB, CHUNK), jnp.int32),
            pltpu.VMEM((NBUF, CHUNK, C), jnp.float32),
            pltpu.VMEM((CHUNK,), jnp.float32),
            pltpu.VMEM_SHARED((HPAD + 8, C), jnp.float32),
            pltpu.VMEM_SHARED((CROWS,), jnp.float32),
            pltpu.SemaphoreType.DMA((NBUF,)),
            pltpu.SemaphoreType.DMA((NBUF,)),
            pltpu.SemaphoreType.DMA((NBUF,)),
        ],
    )
    def sc_kernel(ytab, gat3, sct4, zrows, zdeg,
                  msg_out, deg_out,
                  gatb, sctb, rowsb, onesb, acc, cnt1, gsem, ssem, csem):
        c = lax.axis_index("c")
        s = lax.axis_index("s")

        pltpu.sync_copy(zrows, acc.at[pl.ds(s * RPT, RPT)])
        pltpu.sync_copy(zdeg.at[pl.ds(s * (CROWS // NS), CROWS // NS)],
                        cnt1.at[pl.ds(s * (CROWS // NS), CROWS // NS)])
        for k in range(CHUNK // LANES):
            onesb[pl.ds(k * LANES, LANES)] = jnp.full((LANES,), 1.0, jnp.float32)
        pltpu.sync_copy(gat3.at[s], gatb)
        pltpu.sync_copy(sct4.at[c, s], sctb)

        @pl.when(s == 0)
        def _():
            pltpu.sync_copy(zrows.at[pl.ds(0, 8)], acc.at[pl.ds(HPAD, 8)])

        plsc.subcore_barrier()

        def fire(i, k):
            pltpu.async_copy(ytab.at[gatb.at[i]], rowsb.at[k], gsem.at[k])

        for k in range(NBUF):
            fire(k, k)

        def chunk_body(i, carry):
            k = lax.rem(i, NBUF)
            kp = lax.rem(i + NBUF - 1, NBUF)
            pltpu.make_async_copy(ytab.at[gatb.at[i]], rowsb.at[k],
                                  gsem.at[k]).wait()

            @pl.when(i >= 1)
            def _():
                pltpu.make_async_copy(rowsb.at[kp], acc.at[sctb.at[i - 1]],
                                      ssem.at[kp]).wait()
                pltpu.make_async_copy(onesb, cnt1.at[sctb.at[i - 1]],
                                      csem.at[kp]).wait()

            @pl.when((i >= 1) & (i + NBUF - 1 < NB))
            def _():
                fire(i + NBUF - 1, kp)

            pltpu.async_copy(rowsb.at[k], acc.at[sctb.at[i]], ssem.at[k],
                             add=True)
            pltpu.async_copy(onesb, cnt1.at[sctb.at[i]], csem.at[k],
                             add=True)
            return carry

        lax.fori_loop(0, NB, chunk_body, 0)
        kl = (NB - 1) % NBUF
        pltpu.make_async_copy(rowsb.at[kl], acc.at[sctb.at[NB - 1]],
                              ssem.at[kl]).wait()
        pltpu.make_async_copy(onesb, cnt1.at[sctb.at[NB - 1]],
                              csem.at[kl]).wait()
        plsc.subcore_barrier()

        pltpu.sync_copy(acc.at[pl.ds(s * RPT, RPT)], msg_out.at[c, s])
        pltpu.sync_copy(cnt1.at[pl.ds(s * (CROWS // NS), CROWS // NS)],
                        deg_out.at[c, s])

    return sc_kernel


def _tc_build_y(N, C, R):
    def body(xb, wt, bt, out):
        out[:] = jnp.dot(xb[:], wt[0], preferred_element_type=jnp.float32) + bt[0]

    return pl.pallas_call(
        body,
        grid=(2, N // R),
        in_specs=[
            pl.BlockSpec((R, C), lambda t, j: (j, 0)),
            pl.BlockSpec((1, C, C), lambda t, j: (t, 0, 0)),
            pl.BlockSpec((1, 1, C), lambda t, j: (t, 0, 0)),
        ],
        out_specs=pl.BlockSpec((R, C), lambda t, j: (t * (N // R) + j, 0)),
        out_shape=jax.ShapeDtypeStruct((2 * N, C), jnp.float32),
    )


def _tc_finish(N, C, R):
    def body(m, xb, dg, wself, bs, out):
        acc = jnp.dot(xb[:], wself[:], preferred_element_type=jnp.float32)
        acc += m[:] + bs[:]
        deg = jnp.where(dg[:] == 0.0, 1.0, dg[:])
        out[:] = jnp.maximum(acc, 0.0) / deg

    row = lambda i: (i, 0)
    full = lambda i: (0, 0)
    return pl.pallas_call(
        body,
        grid=(N // R,),
        in_specs=[
            pl.BlockSpec((R, C), row), pl.BlockSpec((R, C), row),
            pl.BlockSpec((R, 1), row),
            pl.BlockSpec((C, C), full), pl.BlockSpec((1, C), full),
        ],
        out_specs=pl.BlockSpec((R, C), row),
        out_shape=jax.ShapeDtypeStruct((N, C), jnp.float32),
    )


def kernel(x, edge_index, edge_types, W_e0, b_e0, W_e1, b_e1, W_self, b_self):
    N, C = x.shape
    E = edge_index.shape[1]
    HPAD = -(-N // (NC * NS * 8)) * (NS * 8)
    RPT = HPAD // NS
    EP = -(-E // (NS * CHUNK)) * (NS * CHUNK)
    NB = EP // (NS * CHUNK)

    wt = jnp.stack([W_e0.T, W_e1.T])
    bt = jnp.stack([b_e0, b_e1]).reshape(2, 1, C)
    ytab = _tc_build_y(N, C, 2000)(x, wt, bt)

    src = edge_index[0]
    dst = edge_index[1]
    gat = edge_types * N + src
    gat = jnp.concatenate([gat, jnp.zeros((EP - E,), jnp.int32)])
    trash = jnp.full((EP - E,), HPAD, jnp.int32)
    scts = []
    for cc in range(NC):
        local = dst - cc * HPAD
        owned = (local >= 0) & (local < HPAD)
        sct = jnp.where(owned, local, HPAD).astype(jnp.int32)
        scts.append(jnp.concatenate([sct, trash]))
    gat3 = gat.reshape(NS, NB, CHUNK)
    sct4 = jnp.stack(scts).reshape(NC, NS, NB, CHUNK)

    zrows = jnp.zeros((RPT, C), jnp.float32)
    zdeg = jnp.zeros((CROWS,), jnp.float32)

    msg, degc = _sc_segment_sum(N, EP, C, NB)(ytab, gat3, sct4, zrows, zdeg)
    msg = msg.reshape(NC * HPAD, C)[:N]
    deg = degc.reshape(NC, CROWS)[:, :HPAD].reshape(NC * HPAD)[:N].reshape(N, 1)

    out = _tc_finish(N, C, 2000)(msg, x, deg, W_self.T, b_self.reshape(1, C))
    return out


# speedup vs baseline: 1.3612x; 1.3612x over previous
import functools

import jax
import jax.numpy as jnp
from jax import lax
from jax.experimental import pallas as pl
from jax.experimental.pallas import tpu as pltpu
from jax.experimental.pallas import tpu_sc as plsc

NC = 2
NS = 16
LANES = 16
CHUNK = 128
CROWS = 10240
NBUF = 2
IBUF = 4


def _sc_segment_sum(N, EP, C, NB):
    NPAD = -(-N // (NS * 8)) * (NS * 8)
    RPT = NPAD // NS
    CPT = CROWS // NS

    @functools.partial(
        pl.kernel,
        out_type=(
            jax.ShapeDtypeStruct((NC, NS, RPT, C), jnp.float32),
            jax.ShapeDtypeStruct((NC, NS, CPT), jnp.float32),
        ),
        mesh=plsc.VectorSubcoreMesh(core_axis_name="c", subcore_axis_name="s"),
        scratch_types=[
            pltpu.VMEM((IBUF, CHUNK), jnp.int32),
            pltpu.VMEM((NB, CHUNK), jnp.int32),
            pltpu.VMEM((NBUF, CHUNK, C), jnp.float32),
            pltpu.VMEM((CHUNK,), jnp.float32),
            pltpu.VMEM_SHARED((NPAD, C), jnp.float32),
            pltpu.VMEM_SHARED((CROWS,), jnp.float32),
            pltpu.SemaphoreType.DMA((IBUF,)),
            pltpu.SemaphoreType.DMA((NBUF,)),
            pltpu.SemaphoreType.DMA((NBUF,)),
            pltpu.SemaphoreType.DMA((NBUF,)),
        ],
    )
    def sc_kernel(ytab, gat3, sct3, zrows, zdeg,
                  msg_out, deg_out,
                  gatb, sctb, rowsb, onesb, acc, cnt1,
                  isem, gsem, ssem, csem):
        c = lax.axis_index("c")
        s = lax.axis_index("s")
        w = c * NS + s

        pltpu.sync_copy(zrows, acc.at[pl.ds(s * RPT, RPT)])
        pltpu.sync_copy(zdeg.at[pl.ds(s * CPT, CPT)],
                        cnt1.at[pl.ds(s * CPT, CPT)])
        for k in range(CHUNK // LANES):
            onesb[pl.ds(k * LANES, LANES)] = jnp.full((LANES,), 1.0,
                                                      jnp.float32)
        pltpu.sync_copy(sct3.at[w], sctb)
        plsc.subcore_barrier()

        pltpu.sync_copy(gat3.at[w, 0], gatb.at[0])
        pltpu.sync_copy(gat3.at[w, 1], gatb.at[1])
        pltpu.async_copy(ytab.at[gatb.at[0]], rowsb.at[0], gsem.at[0])
        pltpu.async_copy(ytab.at[gatb.at[1]], rowsb.at[1], gsem.at[1])

        def chunk_body(i, carry):
            k = lax.rem(i, NBUF)
            j = 1 - k
            ri = lax.rem(i, IBUF)
            r1 = lax.rem(i + 1, IBUF)
            r2 = lax.rem(i + 2, IBUF)

            pltpu.make_async_copy(ytab.at[gatb.at[ri]], rowsb.at[k],
                                  gsem.at[k]).wait()

            @pl.when(i >= 1)
            def _():
                pltpu.make_async_copy(rowsb.at[j], acc.at[sctb.at[i - 1]],
                                      ssem.at[j]).wait()
                pltpu.make_async_copy(onesb, cnt1.at[sctb.at[i - 1]],
                                      csem.at[j]).wait()

            @pl.when((i >= 1) & (i + 1 < NB))
            def _():
                pltpu.make_async_copy(gat3.at[w, i + 1], gatb.at[r1],
                                      isem.at[r1]).wait()
                pltpu.async_copy(ytab.at[gatb.at[r1]], rowsb.at[j],
                                 gsem.at[j])

            @pl.when(i + 2 < NB)
            def _():
                pltpu.async_copy(gat3.at[w, i + 2], gatb.at[r2],
                                 isem.at[r2])

            pltpu.async_copy(rowsb.at[k], acc.at[sctb.at[i]], ssem.at[k],
                             add=True)
            pltpu.async_copy(onesb, cnt1.at[sctb.at[i]], csem.at[k],
                             add=True)
            return carry

        lax.fori_loop(0, NB, chunk_body, 0)
        kl = (NB - 1) % NBUF
        pltpu.make_async_copy(rowsb.at[kl], acc.at[sctb.at[NB - 1]],
                              ssem.at[kl]).wait()
        pltpu.make_async_copy(onesb, cnt1.at[sctb.at[NB - 1]],
                              csem.at[kl]).wait()
        plsc.subcore_barrier()

        pltpu.sync_copy(acc.at[pl.ds(s * RPT, RPT)], msg_out.at[c, s])
        pltpu.sync_copy(cnt1.at[pl.ds(s * CPT, CPT)], deg_out.at[c, s])

    return sc_kernel


def _tc_build_y(N, C, R):
    def body(xb, wt, bt, out):
        out[:] = jnp.dot(xb[:], wt[0], preferred_element_type=jnp.float32) + bt[0]

    return pl.pallas_call(
        body,
        grid=(2, N // R),
        in_specs=[
            pl.BlockSpec((R, C), lambda t, j: (j, 0)),
            pl.BlockSpec((1, C, C), lambda t, j: (t, 0, 0)),
            pl.BlockSpec((1, 1, C), lambda t, j: (t, 0, 0)),
        ],
        out_specs=pl.BlockSpec((R, C), lambda t, j: (t * (N // R) + j, 0)),
        out_shape=jax.ShapeDtypeStruct((2 * N, C), jnp.float32),
    )


def _tc_finish(N, C, R):
    def body(m0, m1, xb, d0, d1, wself, bs, out):
        acc = jnp.dot(xb[:], wself[:], preferred_element_type=jnp.float32)
        acc += m0[:] + m1[:] + bs[:]
        deg = d0[:] + d1[:]
        deg = jnp.where(deg == 0.0, 1.0, deg)
        out[:] = jnp.maximum(acc, 0.0) / deg

    row = lambda i: (i, 0)
    full = lambda i: (0, 0)
    return pl.pallas_call(
        body,
        grid=(N // R,),
        in_specs=[
            pl.BlockSpec((R, C), row), pl.BlockSpec((R, C), row),
            pl.BlockSpec((R, C), row),
            pl.BlockSpec((R, 1), row), pl.BlockSpec((R, 1), row),
            pl.BlockSpec((C, C), full), pl.BlockSpec((1, C), full),
        ],
        out_specs=pl.BlockSpec((R, C), row),
        out_shape=jax.ShapeDtypeStruct((N, C), jnp.float32),
    )


def kernel(x, edge_index, edge_types, W_e0, b_e0, W_e1, b_e1, W_self, b_self):
    N, C = x.shape
    E = edge_index.shape[1]
    NPAD = -(-N // (NS * 8)) * (NS * 8)
    RPT = NPAD // NS
    W = NC * NS
    EP = -(-E // (W * CHUNK)) * (W * CHUNK)
    NB = EP // (W * CHUNK)

    wt = jnp.stack([W_e0.T, W_e1.T])
    bt = jnp.stack([b_e0, b_e1]).reshape(2, 1, C)
    ytab = _tc_build_y(N, C, 2000)(x, wt, bt)

    src = edge_index[0]
    dst = edge_index[1]
    gat = jnp.concatenate([edge_types * N + src,
                           jnp.zeros((EP - E,), jnp.int32)])
    sct = jnp.concatenate([dst, jnp.full((EP - E,), N, jnp.int32)])
    gat3 = gat.reshape(W, NB, CHUNK)
    sct3 = sct.reshape(W, NB, CHUNK)

    zrows = jnp.zeros((RPT, C), jnp.float32)
    zdeg = jnp.zeros((CROWS,), jnp.float32)

    msg, degc = _sc_segment_sum(N, EP, C, NB)(ytab, gat3, sct3, zrows, zdeg)
    msg = msg.reshape(NC, NPAD, C)
    degc = degc.reshape(NC, CROWS)

    out = _tc_finish(N, C, 2000)(
        msg[0, :N], msg[1, :N], x,
        degc[0, :N].reshape(N, 1), degc[1, :N].reshape(N, 1),
        W_self.T, b_self.reshape(1, C))
    return out
